# Initial kernel scaffold; baseline (speedup 1.0000x reference)
import jax
import jax.numpy as jnp
from jax.experimental import pallas as pl


def kernel(feature, edge_index, edge_weight, feature2, edge_index2, edge_weight2, Wx0, bx0, Wx1, bx1, Wa0, Wa1, ba1, Wg0, bg0, Wg1, bg1, Wz):
    raise NotImplementedError("write your pallas kernel here")


# speedup vs baseline: 2.8982x; 2.8982x over previous
import functools

import jax
import jax.numpy as jnp
from jax import lax
from jax.experimental import pallas as pl
from jax.experimental.pallas import tpu as pltpu
from jax.experimental.pallas import tpu_sc as plsc

N = 10000
E = 320000
IN = 128
H = 128
OUT = 40

NC = 2
NS = 16
NW = NC * NS
EPW = E // NW
C = 80
NCHUNK = EPW // C
RPT = 624
NTAIL = N - RPT * NS
RZ = 48
LANES = 16


def _seg_kernel(table_hbm, src_hbm, dst_hbm, w_hbm, out_hbm,
                src_v, dst_v, w_v, rows_v, zbuf, acc, sem):
  c = lax.axis_index("c")
  s = lax.axis_index("s")
  wid = c * NS + s

  zero16 = jnp.zeros((LANES,), jnp.float32)

  def zero_body(i, carry):
    for j in range(H // LANES):
      zbuf[i, pl.ds(j * LANES, LANES)] = zero16
    return carry

  lax.fori_loop(0, RZ, zero_body, 0)
  row0 = pl.multiple_of(s * RPT, 8)
  for k in range(RPT // RZ):
    pltpu.sync_copy(zbuf, acc.at[pl.ds(pl.multiple_of(row0 + k * RZ, 8), RZ)])

  @pl.when(s == NS - 1)
  def _zero_tail():
    pltpu.sync_copy(zbuf.at[pl.ds(0, NTAIL)], acc.at[pl.ds(N - NTAIL, NTAIL)])

  plsc.subcore_barrier()

  base = wid * EPW

  def chunk_body(k, carry):
    off = pl.multiple_of(base + k * C, 8)
    pltpu.sync_copy(src_hbm.at[pl.ds(off, C)], src_v)
    pltpu.sync_copy(dst_hbm.at[pl.ds(off, C)], dst_v)
    pltpu.sync_copy(w_hbm.at[pl.ds(off, C)], w_v)
    pltpu.async_copy(table_hbm.at[src_v], rows_v, sem).wait()

    def edge_body(e, ecarry):
      w16 = plsc.load_gather(w_v, [jnp.full((LANES,), e, jnp.int32)])
      for j in range(H // LANES):
        sl = pl.ds(j * LANES, LANES)
        rows_v[e, sl] = rows_v[e, sl] * w16
      return ecarry

    lax.fori_loop(0, C, edge_body, 0)
    pltpu.sync_copy(rows_v, acc.at[dst_v], add=True)
    return carry

  lax.fori_loop(0, NCHUNK, chunk_body, 0)
  plsc.subcore_barrier()
  pltpu.sync_copy(acc.at[pl.ds(row0, RPT)], out_hbm.at[c, pl.ds(row0, RPT)])

  @pl.when(s == NS - 1)
  def _write_tail():
    pltpu.sync_copy(acc.at[pl.ds(N - NTAIL, NTAIL)],
                    out_hbm.at[c, pl.ds(N - NTAIL, NTAIL)])


def _segment_sum_sc(table, src, dst, w):
  mesh = plsc.VectorSubcoreMesh(core_axis_name="c", subcore_axis_name="s",
                                num_cores=NC, num_subcores=NS)
  fn = pl.kernel(
      _seg_kernel,
      out_type=jax.ShapeDtypeStruct((NC, N, H), jnp.float32),
      mesh=mesh,
      compiler_params=pltpu.CompilerParams(needs_layout_passes=False),
      scratch_types=[
          pltpu.VMEM((C,), jnp.int32),
          pltpu.VMEM((C,), jnp.int32),
          pltpu.VMEM((C,), jnp.float32),
          pltpu.VMEM((C, H), jnp.float32),
          pltpu.VMEM((RZ, H), jnp.float32),
          pltpu.VMEM_SHARED((N, H), jnp.float32),
          pltpu.SemaphoreType.DMA,
      ],
  )
  return fn(table, src, dst, w)


BM = 512
GRID = (N + BM - 1) // BM


def _mm_a_body(f_ref, f2_ref, wx0_ref, bx0_ref, wx1_ref, bx1_ref, wg0_ref,
               xh_ref, h1_ref):
  f = f_ref[...]
  h = jnp.maximum(
      jnp.dot(f, wx0_ref[...], preferred_element_type=jnp.float32)
      + bx0_ref[...], 0.0)
  xh_ref[...] = (jnp.dot(h, wx1_ref[...], preferred_element_type=jnp.float32)
                 + bx1_ref[...])
  h1_ref[...] = jnp.dot(f2_ref[...], wg0_ref[...],
                        preferred_element_type=jnp.float32)


def _mm_b_body(ap_ref, gp_ref, wa1_ref, ba1_ref, bg0_ref, wg1_ref,
               ah_ref, h2_ref):
  a = jnp.maximum(ap_ref[0] + ap_ref[1], 0.0)
  ah_ref[...] = (jnp.dot(a, wa1_ref[...], preferred_element_type=jnp.float32)
                 + ba1_ref[...])
  h1r = jnp.maximum(gp_ref[0] + gp_ref[1] + bg0_ref[...], 0.0)
  h2_ref[...] = jnp.dot(h1r, wg1_ref[...], preferred_element_type=jnp.float32)


def _mm_c_body(xh_ref, ah_ref, op_ref, bg1_ref, wz_ref, out1_ref, logp_ref):
  out1 = op_ref[0] + op_ref[1] + bg1_ref[...]
  out1_ref[...] = out1
  z = xh_ref[...] + ah_ref[...] + out1
  zz = jnp.dot(z, wz_ref[...], preferred_element_type=jnp.float32)
  m = jnp.max(zz, axis=1, keepdims=True)
  ez = jnp.exp(zz - m)
  lse = jnp.log(jnp.sum(ez, axis=1, keepdims=True)) + m
  logp_ref[...] = zz - lse


def _row_spec(shape):
  nd = len(shape)
  return pl.BlockSpec((BM,) + shape[1:], lambda i: (i,) + (0,) * (nd - 1))


def _full_spec(shape):
  nd = len(shape)
  return pl.BlockSpec(shape, lambda i: (0,) * nd)


def _part_spec(shape):
  return pl.BlockSpec((2, BM, shape[2]), lambda i: (0, i, 0))


def kernel(feature, edge_index, edge_weight, feature2, edge_index2,
           edge_weight2, Wx0, bx0, Wx1, bx1, Wa0, Wa1, ba1, Wg0, bg0, Wg1,
           bg1, Wz):
  src = edge_index[0]
  dst = edge_index[1]
  s2 = edge_index2[0]
  d2 = edge_index2[1]
  bx0r = bx0.reshape(1, H)
  bx1r = bx1.reshape(1, H)
  ba1r = ba1.reshape(1, H)
  bg0r = bg0.reshape(1, H)
  bg1r = bg1.reshape(1, H)

  a_p = _segment_sum_sc(Wa0, src, dst, edge_weight)

  x_h, h1 = pl.pallas_call(
      _mm_a_body,
      grid=(GRID,),
      in_specs=[
          _row_spec((N, IN)), _row_spec((N, IN)),
          _full_spec((IN, H)), _full_spec((1, H)),
          _full_spec((H, H)), _full_spec((1, H)),
          _full_spec((IN, H)),
      ],
      out_specs=[_row_spec((N, H)), _row_spec((N, H))],
      out_shape=[
          jax.ShapeDtypeStruct((N, H), jnp.float32),
          jax.ShapeDtypeStruct((N, H), jnp.float32),
      ],
  )(feature, feature2, Wx0, bx0r, Wx1, bx1r, Wg0)

  g_p = _segment_sum_sc(h1, s2, d2, edge_weight2)

  a_h, h2 = pl.pallas_call(
      _mm_b_body,
      grid=(GRID,),
      in_specs=[
          _part_spec((2, N, H)), _part_spec((2, N, H)),
          _full_spec((H, H)), _full_spec((1, H)),
          _full_spec((1, H)), _full_spec((H, H)),
      ],
      out_specs=[_row_spec((N, H)), _row_spec((N, H))],
      out_shape=[
          jax.ShapeDtypeStruct((N, H), jnp.float32),
          jax.ShapeDtypeStruct((N, H), jnp.float32),
      ],
  )(a_p, g_p, Wa1, ba1r, bg0r, Wg1)

  o_p = _segment_sum_sc(h2, s2, d2, edge_weight2)

  output1, logp = pl.pallas_call(
      _mm_c_body,
      grid=(GRID,),
      in_specs=[
          _row_spec((N, H)), _row_spec((N, H)), _part_spec((2, N, H)),
          _full_spec((1, H)), _full_spec((H, OUT)),
      ],
      out_specs=[_row_spec((N, H)), _row_spec((N, OUT))],
      out_shape=[
          jax.ShapeDtypeStruct((N, H), jnp.float32),
          jax.ShapeDtypeStruct((N, OUT), jnp.float32),
      ],
  )(x_h, a_h, o_p, bg1r, Wz)

  return (x_h, a_h, output1, logp)


# speedup vs baseline: 5.7954x; 1.9996x over previous
import functools

import jax
import jax.numpy as jnp
from jax import lax
from jax.experimental import pallas as pl
from jax.experimental.pallas import tpu as pltpu
from jax.experimental.pallas import tpu_sc as plsc

N = 10000
E = 320000
IN = 128
H = 128
OUT = 40

NC = 2
NS = 16
NW = NC * NS
EPW = E // NW
C = 125
NCHUNK = EPW // C
G = 16
NGROUP = NCHUNK // G
UNROLL = 5
RPT = 624
NTAIL = N - RPT * NS
RZ = 120
LANES = 16


def _seg_kernel(table_hbm, src_hbm, dst_hbm, w_hbm, out_hbm,
                src_v, dst_v, w_v, rows0_v, rows1_v, acc,
                gsem0, gsem1, ssem0, ssem1, isem):
  c = lax.axis_index("c")
  s = lax.axis_index("s")
  wid = c * NS + s

  def _stage(gi, buf):
    crow = pl.multiple_of(wid * NCHUNK + gi * G, 8)
    return (pltpu.async_copy(src_hbm.at[pl.ds(crow, G)], src_v.at[buf], isem),
            pltpu.async_copy(dst_hbm.at[pl.ds(crow, G)], dst_v.at[buf], isem),
            pltpu.async_copy(w_hbm.at[pl.ds(crow, G)], w_v.at[buf], isem))

  stage0 = _stage(0, 0)

  zero16 = jnp.zeros((LANES,), jnp.float32)

  def zero_body(i, carry):
    for j in range(H // LANES):
      rows0_v[i, pl.ds(j * LANES, LANES)] = zero16
    return carry

  lax.fori_loop(0, C, zero_body, 0)
  row0 = pl.multiple_of(s * RPT, 8)
  for k in range(RPT // RZ):
    pltpu.sync_copy(rows0_v.at[pl.ds(0, RZ)],
                    acc.at[pl.ds(pl.multiple_of(row0 + k * RZ, 8), RZ)])
  pltpu.sync_copy(rows0_v.at[pl.ds(0, RPT - (RPT // RZ) * RZ)],
                  acc.at[pl.ds(pl.multiple_of(row0 + (RPT // RZ) * RZ, 8),
                               RPT - (RPT // RZ) * RZ)])

  @pl.when(s == NS - 1)
  def _zero_tail():
    pltpu.sync_copy(rows0_v.at[pl.ds(0, NTAIL)],
                    acc.at[pl.ds(N - NTAIL, NTAIL)])

  plsc.subcore_barrier()

  def _scale(rows_v, buf, k):
    def edge_body(eg, ecarry):
      for u in range(UNROLL):
        e = eg * UNROLL + u
        w16 = plsc.load_gather(
            w_v, [jnp.full((LANES,), buf, jnp.int32),
                  jnp.full((LANES,), k, jnp.int32),
                  jnp.full((LANES,), e, jnp.int32)])
        for j in range(H // LANES):
          sl = pl.ds(j * LANES, LANES)
          rows_v[e, sl] = rows_v[e, sl] * w16
      return ecarry

    lax.fori_loop(0, C // UNROLL, edge_body, 0)

  for gi in range(NGROUP):
    buf = gi % 2
    for cp in stage0:
      cp.wait()
    if gi + 1 < NGROUP:
      stage0 = _stage(gi + 1, 1 - buf)

    def pair_body(i, carry, buf=buf):
      a = i * 2
      b = a + 1
      g_a = pltpu.async_copy(table_hbm.at[src_v.at[buf, a]], rows0_v, gsem0)
      g_b = pltpu.async_copy(table_hbm.at[src_v.at[buf, b]], rows1_v, gsem1)
      g_a.wait()
      _scale(rows0_v, buf, a)
      s_a = pltpu.async_copy(rows0_v, acc.at[dst_v.at[buf, a]], ssem0,
                             add=True)
      g_b.wait()
      _scale(rows1_v, buf, b)
      s_b = pltpu.async_copy(rows1_v, acc.at[dst_v.at[buf, b]], ssem1,
                             add=True)
      s_a.wait()
      s_b.wait()
      return carry

    lax.fori_loop(0, G // 2, pair_body, 0)

  plsc.subcore_barrier()
  pltpu.sync_copy(acc.at[pl.ds(row0, RPT)], out_hbm.at[c, pl.ds(row0, RPT)])

  @pl.when(s == NS - 1)
  def _write_tail():
    pltpu.sync_copy(acc.at[pl.ds(N - NTAIL, NTAIL)],
                    out_hbm.at[c, pl.ds(N - NTAIL, NTAIL)])


def _segment_sum_sc(table, src, dst, w):
  mesh = plsc.VectorSubcoreMesh(core_axis_name="c", subcore_axis_name="s",
                                num_cores=NC, num_subcores=NS)
  fn = pl.kernel(
      _seg_kernel,
      out_type=jax.ShapeDtypeStruct((NC, N, H), jnp.float32),
      mesh=mesh,
      compiler_params=pltpu.CompilerParams(needs_layout_passes=False),
      scratch_types=[
          pltpu.VMEM((2, G, C), jnp.int32),
          pltpu.VMEM((2, G, C), jnp.int32),
          pltpu.VMEM((2, G, C), jnp.float32),
          pltpu.VMEM((C, H), jnp.float32),
          pltpu.VMEM((C, H), jnp.float32),
          pltpu.VMEM_SHARED((N, H), jnp.float32),
          pltpu.SemaphoreType.DMA,
          pltpu.SemaphoreType.DMA,
          pltpu.SemaphoreType.DMA,
          pltpu.SemaphoreType.DMA,
          pltpu.SemaphoreType.DMA,
      ],
  )
  src2 = src.reshape(NW * NCHUNK, C)
  dst2 = dst.reshape(NW * NCHUNK, C)
  w2 = w.reshape(NW * NCHUNK, C)
  return fn(table, src2, dst2, w2)


BM = 512
GRID = (N + BM - 1) // BM


def _mm_a_body(f_ref, f2_ref, wx0_ref, bx0_ref, wx1_ref, bx1_ref, wg0_ref,
               xh_ref, h1_ref):
  f = f_ref[...]
  h = jnp.maximum(
      jnp.dot(f, wx0_ref[...], preferred_element_type=jnp.float32)
      + bx0_ref[...], 0.0)
  xh_ref[...] = (jnp.dot(h, wx1_ref[...], preferred_element_type=jnp.float32)
                 + bx1_ref[...])
  h1_ref[...] = jnp.dot(f2_ref[...], wg0_ref[...],
                        preferred_element_type=jnp.float32)


def _mm_b_body(ap_ref, gp_ref, wa1_ref, ba1_ref, bg0_ref, wg1_ref,
               ah_ref, h2_ref):
  a = jnp.maximum(ap_ref[0] + ap_ref[1], 0.0)
  ah_ref[...] = (jnp.dot(a, wa1_ref[...], preferred_element_type=jnp.float32)
                 + ba1_ref[...])
  h1r = jnp.maximum(gp_ref[0] + gp_ref[1] + bg0_ref[...], 0.0)
  h2_ref[...] = jnp.dot(h1r, wg1_ref[...], preferred_element_type=jnp.float32)


def _mm_c_body(xh_ref, ah_ref, op_ref, bg1_ref, wz_ref, out1_ref, logp_ref):
  out1 = op_ref[0] + op_ref[1] + bg1_ref[...]
  out1_ref[...] = out1
  z = xh_ref[...] + ah_ref[...] + out1
  zz = jnp.dot(z, wz_ref[...], preferred_element_type=jnp.float32)
  m = jnp.max(zz, axis=1, keepdims=True)
  ez = jnp.exp(zz - m)
  lse = jnp.log(jnp.sum(ez, axis=1, keepdims=True)) + m
  logp_ref[...] = zz - lse


def _row_spec(shape):
  nd = len(shape)
  return pl.BlockSpec((BM,) + shape[1:], lambda i: (i,) + (0,) * (nd - 1))


def _full_spec(shape):
  nd = len(shape)
  return pl.BlockSpec(shape, lambda i: (0,) * nd)


def _part_spec(shape):
  return pl.BlockSpec((2, BM, shape[2]), lambda i: (0, i, 0))


def kernel(feature, edge_index, edge_weight, feature2, edge_index2,
           edge_weight2, Wx0, bx0, Wx1, bx1, Wa0, Wa1, ba1, Wg0, bg0, Wg1,
           bg1, Wz):
  src = edge_index[0]
  dst = edge_index[1]
  s2 = edge_index2[0]
  d2 = edge_index2[1]
  bx0r = bx0.reshape(1, H)
  bx1r = bx1.reshape(1, H)
  ba1r = ba1.reshape(1, H)
  bg0r = bg0.reshape(1, H)
  bg1r = bg1.reshape(1, H)

  a_p = _segment_sum_sc(Wa0, src, dst, edge_weight)

  x_h, h1 = pl.pallas_call(
      _mm_a_body,
      grid=(GRID,),
      in_specs=[
          _row_spec((N, IN)), _row_spec((N, IN)),
          _full_spec((IN, H)), _full_spec((1, H)),
          _full_spec((H, H)), _full_spec((1, H)),
          _full_spec((IN, H)),
      ],
      out_specs=[_row_spec((N, H)), _row_spec((N, H))],
      out_shape=[
          jax.ShapeDtypeStruct((N, H), jnp.float32),
          jax.ShapeDtypeStruct((N, H), jnp.float32),
      ],
  )(feature, feature2, Wx0, bx0r, Wx1, bx1r, Wg0)

  g_p = _segment_sum_sc(h1, s2, d2, edge_weight2)

  a_h, h2 = pl.pallas_call(
      _mm_b_body,
      grid=(GRID,),
      in_specs=[
          _part_spec((2, N, H)), _part_spec((2, N, H)),
          _full_spec((H, H)), _full_spec((1, H)),
          _full_spec((1, H)), _full_spec((H, H)),
      ],
      out_specs=[_row_spec((N, H)), _row_spec((N, H))],
      out_shape=[
          jax.ShapeDtypeStruct((N, H), jnp.float32),
          jax.ShapeDtypeStruct((N, H), jnp.float32),
      ],
  )(a_p, g_p, Wa1, ba1r, bg0r, Wg1)

  o_p = _segment_sum_sc(h2, s2, d2, edge_weight2)

  output1, logp = pl.pallas_call(
      _mm_c_body,
      grid=(GRID,),
      in_specs=[
          _row_spec((N, H)), _row_spec((N, H)), _part_spec((2, N, H)),
          _full_spec((1, H)), _full_spec((H, OUT)),
      ],
      out_specs=[_row_spec((N, H)), _row_spec((N, OUT))],
      out_shape=[
          jax.ShapeDtypeStruct((N, H), jnp.float32),
          jax.ShapeDtypeStruct((N, OUT), jnp.float32),
      ],
  )(x_h, a_h, o_p, bg1r, Wz)

  return (x_h, a_h, output1, logp)


# speedup vs baseline: 7.5442x; 1.3017x over previous
import functools

import jax
import jax.numpy as jnp
from jax import lax
from jax.experimental import pallas as pl
from jax.experimental.pallas import tpu as pltpu
from jax.experimental.pallas import tpu_sc as plsc

N = 10000
E = 320000
IN = 128
H = 128
OUT = 40

NC = 2
NS = 16
NW = NC * NS
EPW = E // NW
C = 50
NCHUNK = EPW // C
RB = 3
UNROLL = 5
RPT = 624
NTAIL = N - RPT * NS
RZ = 48
LANES = 16


def _seg_kernel(table_hbm, comb_hbm, w_hbm, out_hbm,
                comb_v, w_v, rows0_v, rows1_v, rows2_v,
                sr0, sr1, sr2, dr0, dr1, dr2, acc,
                gsem0, gsem1, gsem2, ssem0, ssem1, ssem2, isem):
  c = lax.axis_index("c")
  s = lax.axis_index("s")
  wid = c * NS + s
  rows = (rows0_v, rows1_v, rows2_v)
  sring = (sr0, sr1, sr2)
  dring = (dr0, dr1, dr2)
  gsem = (gsem0, gsem1, gsem2)
  ssem = (ssem0, ssem1, ssem2)

  ebase = pl.multiple_of(wid * EPW, 8)
  i_comb = pltpu.async_copy(comb_hbm.at[pl.ds(ebase, EPW)], comb_v, isem)
  i_w = pltpu.async_copy(w_hbm.at[pl.ds(ebase, EPW)], w_v, isem)

  zero16 = jnp.zeros((LANES,), jnp.float32)

  def zero_body(i, carry):
    for j in range(H // LANES):
      rows0_v[i, pl.ds(j * LANES, LANES)] = zero16
    return carry

  lax.fori_loop(0, RZ, zero_body, 0)
  row0 = pl.multiple_of(s * RPT, 8)
  for k in range(RPT // RZ):
    pltpu.sync_copy(rows0_v.at[pl.ds(0, RZ)],
                    acc.at[pl.ds(pl.multiple_of(row0 + k * RZ, 8), RZ)])

  @pl.when(s == NS - 1)
  def _zero_tail():
    pltpu.sync_copy(rows0_v.at[pl.ds(0, NTAIL)],
                    acc.at[pl.ds(N - NTAIL, NTAIL)])

  i_comb.wait()
  i_w.wait()
  plsc.subcore_barrier()

  def _scale(rows_v, k):
    def edge_body(eg, ecarry):
      for u in range(UNROLL):
        e = eg * UNROLL + u
        w16 = plsc.load_gather(
            w_v, [jnp.full((LANES,), k * C + e, jnp.int32)])
        for j in range(H // LANES):
          sl = pl.ds(j * LANES, LANES)
          rows_v[e, sl] = rows_v[e, sl] * w16
      return ecarry

    lax.fori_loop(0, C // UNROLL, edge_body, 0)

  def _fill_ring(ring, k, is_src):
    base = k * C
    for off in (0, 16, 32, C - LANES):
      packed = comb_v[pl.ds(base + off, LANES)]
      if is_src:
        ring[pl.ds(off, LANES)] = lax.shift_right_logical(packed, 14)
      else:
        ring[pl.ds(off, LANES)] = lax.bitwise_and(
            packed, jnp.full((LANES,), 16383, jnp.int32))

  def _issue_gather(k, b):
    _fill_ring(sring[b], k, True)
    return pltpu.async_copy(table_hbm.at[sring[b]], rows[b], gsem[b])

  def _issue_scatter(k, b):
    _fill_ring(dring[b], k, False)
    return pltpu.async_copy(rows[b], acc.at[dring[b]], ssem[b], add=True)

  def _wait_gather(b):
    pltpu.make_async_copy(table_hbm.at[sring[b]], rows[b], gsem[b]).wait()

  def _wait_scatter(b):
    pltpu.make_async_copy(rows[b], acc.at[dring[b]], ssem[b]).wait()

  def _visit(v, b, head, tail):
    nb = (b + 1) % RB
    if not head:
      _wait_scatter(nb)
    if not tail:
      _issue_gather(v + 1, nb)
    _wait_gather(b)
    _scale(rows[b], v)
    _issue_scatter(v, b)

  _issue_gather(0, 0)
  _visit(0, 0, head=True, tail=False)
  _visit(1, 1, head=True, tail=False)

  def ring_body(j, carry):
    v = j * RB
    for u in range(RB):
      _visit(v + u, u, head=False, tail=False)
    return carry

  _visit(2, 2, head=False, tail=False)
  lax.fori_loop(1, (NCHUNK - 2) // RB, ring_body, 0)
  _visit(NCHUNK - 2, (NCHUNK - 2) % RB, head=False, tail=False)
  _visit(NCHUNK - 1, (NCHUNK - 1) % RB, head=False, tail=True)
  _wait_scatter((NCHUNK - 2) % RB)
  _wait_scatter((NCHUNK - 1) % RB)

  plsc.subcore_barrier()
  pltpu.sync_copy(acc.at[pl.ds(row0, RPT)], out_hbm.at[c, pl.ds(row0, RPT)])

  @pl.when(s == NS - 1)
  def _write_tail():
    pltpu.sync_copy(acc.at[pl.ds(N - NTAIL, NTAIL)],
                    out_hbm.at[c, pl.ds(N - NTAIL, NTAIL)])


def _segment_sum_sc(table, src, dst, w):
  mesh = plsc.VectorSubcoreMesh(core_axis_name="c", subcore_axis_name="s",
                                num_cores=NC, num_subcores=NS)
  fn = pl.kernel(
      _seg_kernel,
      out_type=jax.ShapeDtypeStruct((NC, N, H), jnp.float32),
      mesh=mesh,
      compiler_params=pltpu.CompilerParams(needs_layout_passes=False),
      scratch_types=[
          pltpu.VMEM((EPW,), jnp.int32),
          pltpu.VMEM((EPW,), jnp.float32),
          pltpu.VMEM((C, H), jnp.float32),
          pltpu.VMEM((C, H), jnp.float32),
          pltpu.VMEM((C, H), jnp.float32),
          pltpu.VMEM((C,), jnp.int32),
          pltpu.VMEM((C,), jnp.int32),
          pltpu.VMEM((C,), jnp.int32),
          pltpu.VMEM((C,), jnp.int32),
          pltpu.VMEM((C,), jnp.int32),
          pltpu.VMEM((C,), jnp.int32),
          pltpu.VMEM_SHARED((N, H), jnp.float32),
          pltpu.SemaphoreType.DMA,
          pltpu.SemaphoreType.DMA,
          pltpu.SemaphoreType.DMA,
          pltpu.SemaphoreType.DMA,
          pltpu.SemaphoreType.DMA,
          pltpu.SemaphoreType.DMA,
          pltpu.SemaphoreType.DMA,
      ],
  )
  comb = jnp.left_shift(src, 14) | dst
  return fn(table, comb, w)


BM = 512
GRID = (N + BM - 1) // BM


def _mm_a_body(f_ref, f2_ref, wx0_ref, bx0_ref, wx1_ref, bx1_ref, wg0_ref,
               xh_ref, h1_ref):
  f = f_ref[...]
  h = jnp.maximum(
      jnp.dot(f, wx0_ref[...], preferred_element_type=jnp.float32)
      + bx0_ref[...], 0.0)
  xh_ref[...] = (jnp.dot(h, wx1_ref[...], preferred_element_type=jnp.float32)
                 + bx1_ref[...])
  h1_ref[...] = jnp.dot(f2_ref[...], wg0_ref[...],
                        preferred_element_type=jnp.float32)


def _mm_b_body(ap_ref, gp_ref, wa1_ref, ba1_ref, bg0_ref, wg1_ref,
               ah_ref, h2_ref):
  a = jnp.maximum(ap_ref[0] + ap_ref[1], 0.0)
  ah_ref[...] = (jnp.dot(a, wa1_ref[...], preferred_element_type=jnp.float32)
                 + ba1_ref[...])
  h1r = jnp.maximum(gp_ref[0] + gp_ref[1] + bg0_ref[...], 0.0)
  h2_ref[...] = jnp.dot(h1r, wg1_ref[...], preferred_element_type=jnp.float32)


def _mm_c_body(xh_ref, ah_ref, op_ref, bg1_ref, wz_ref, out1_ref, logp_ref):
  out1 = op_ref[0] + op_ref[1] + bg1_ref[...]
  out1_ref[...] = out1
  z = xh_ref[...] + ah_ref[...] + out1
  zz = jnp.dot(z, wz_ref[...], preferred_element_type=jnp.float32)
  m = jnp.max(zz, axis=1, keepdims=True)
  ez = jnp.exp(zz - m)
  lse = jnp.log(jnp.sum(ez, axis=1, keepdims=True)) + m
  logp_ref[...] = zz - lse


def _row_spec(shape):
  nd = len(shape)
  return pl.BlockSpec((BM,) + shape[1:], lambda i: (i,) + (0,) * (nd - 1))


def _full_spec(shape):
  nd = len(shape)
  return pl.BlockSpec(shape, lambda i: (0,) * nd)


def _part_spec(shape):
  return pl.BlockSpec((2, BM, shape[2]), lambda i: (0, i, 0))


def kernel(feature, edge_index, edge_weight, feature2, edge_index2,
           edge_weight2, Wx0, bx0, Wx1, bx1, Wa0, Wa1, ba1, Wg0, bg0, Wg1,
           bg1, Wz):
  src = edge_index[0]
  dst = edge_index[1]
  s2 = edge_index2[0]
  d2 = edge_index2[1]
  bx0r = bx0.reshape(1, H)
  bx1r = bx1.reshape(1, H)
  ba1r = ba1.reshape(1, H)
  bg0r = bg0.reshape(1, H)
  bg1r = bg1.reshape(1, H)

  a_p = _segment_sum_sc(Wa0, src, dst, edge_weight)

  x_h, h1 = pl.pallas_call(
      _mm_a_body,
      grid=(GRID,),
      in_specs=[
          _row_spec((N, IN)), _row_spec((N, IN)),
          _full_spec((IN, H)), _full_spec((1, H)),
          _full_spec((H, H)), _full_spec((1, H)),
          _full_spec((IN, H)),
      ],
      out_specs=[_row_spec((N, H)), _row_spec((N, H))],
      out_shape=[
          jax.ShapeDtypeStruct((N, H), jnp.float32),
          jax.ShapeDtypeStruct((N, H), jnp.float32),
      ],
  )(feature, feature2, Wx0, bx0r, Wx1, bx1r, Wg0)

  g_p = _segment_sum_sc(h1, s2, d2, edge_weight2)

  a_h, h2 = pl.pallas_call(
      _mm_b_body,
      grid=(GRID,),
      in_specs=[
          _part_spec((2, N, H)), _part_spec((2, N, H)),
          _full_spec((H, H)), _full_spec((1, H)),
          _full_spec((1, H)), _full_spec((H, H)),
      ],
      out_specs=[_row_spec((N, H)), _row_spec((N, H))],
      out_shape=[
          jax.ShapeDtypeStruct((N, H), jnp.float32),
          jax.ShapeDtypeStruct((N, H), jnp.float32),
      ],
  )(a_p, g_p, Wa1, ba1r, bg0r, Wg1)

  o_p = _segment_sum_sc(h2, s2, d2, edge_weight2)

  output1, logp = pl.pallas_call(
      _mm_c_body,
      grid=(GRID,),
      in_specs=[
          _row_spec((N, H)), _row_spec((N, H)), _part_spec((2, N, H)),
          _full_spec((1, H)), _full_spec((H, OUT)),
      ],
      out_specs=[_row_spec((N, H)), _row_spec((N, OUT))],
      out_shape=[
          jax.ShapeDtypeStruct((N, H), jnp.float32),
          jax.ShapeDtypeStruct((N, OUT), jnp.float32),
      ],
  )(x_h, a_h, o_p, bg1r, Wz)

  return (x_h, a_h, output1, logp)


# speedup vs baseline: 8.3491x; 1.1067x over previous
import functools

import jax
import jax.numpy as jnp
from jax import lax
from jax.experimental import pallas as pl
from jax.experimental.pallas import tpu as pltpu
from jax.experimental.pallas import tpu_sc as plsc

N = 10000
E = 320000
IN = 128
H = 128
OUT = 40

NC = 2
NS = 16
NW = NC * NS
EPW = E // NW
C = 80
NCHUNK = EPW // C
RB = 4
UNROLL = 8
RPT = 624
NTAIL = N - RPT * NS
RZ = 48
LANES = 16


def _seg_kernel(table_hbm, comb_hbm, w_hbm, out_hbm, *refs):
  rows = refs[0:RB]
  sring = refs[RB:2 * RB]
  dring = refs[2 * RB:3 * RB]
  cring = refs[3 * RB:4 * RB]
  wring = refs[4 * RB:5 * RB]
  acc = refs[5 * RB]
  gsem = refs[5 * RB + 1:6 * RB + 1]
  ssem = refs[6 * RB + 1:7 * RB + 1]
  fsem = refs[7 * RB + 1:8 * RB + 1]

  c = lax.axis_index("c")
  s = lax.axis_index("s")
  wid = c * NS + s
  ebase = pl.multiple_of(wid * EPW, 8)

  def _issue_fetch(k, r):
    off = pl.multiple_of(ebase + k * C, 8)
    pltpu.async_copy(comb_hbm.at[pl.ds(off, C)], cring[r], fsem[r])
    pltpu.async_copy(w_hbm.at[pl.ds(off, C)], wring[r], fsem[r])

  def _wait_fetch(r):
    pltpu.make_async_copy(comb_hbm.at[pl.ds(ebase, C)], cring[r],
                          fsem[r]).wait()
    pltpu.make_async_copy(w_hbm.at[pl.ds(ebase, C)], wring[r],
                          fsem[r]).wait()

  def _unpack_src(r):
    for off in range(0, C, LANES):
      sl = pl.ds(off, LANES)
      sring[r][sl] = lax.shift_right_logical(cring[r][sl], 14)

  def _unpack_dst(r):
    m = jnp.full((LANES,), 16383, jnp.int32)
    for off in range(0, C, LANES):
      sl = pl.ds(off, LANES)
      dring[r][sl] = lax.bitwise_and(cring[r][sl], m)

  def _issue_gather(b):
    return pltpu.async_copy(table_hbm.at[sring[b]], rows[b], gsem[b])

  def _issue_scatter(b):
    return pltpu.async_copy(rows[b], acc.at[dring[b]], ssem[b], add=True)

  def _wait_gather(b):
    pltpu.make_async_copy(table_hbm.at[sring[b]], rows[b], gsem[b]).wait()

  def _wait_scatter(b):
    pltpu.make_async_copy(rows[b], acc.at[dring[b]], ssem[b]).wait()

  _issue_fetch(0, 0)
  _issue_fetch(1, 1)

  zero16 = jnp.zeros((LANES,), jnp.float32)

  def zero_body(i, carry):
    for j in range(H // LANES):
      rows[RB - 1][i, pl.ds(j * LANES, LANES)] = zero16
    return carry

  lax.fori_loop(0, RZ, zero_body, 0)
  row0 = pl.multiple_of(s * RPT, 8)
  for k in range(RPT // RZ):
    pltpu.sync_copy(rows[RB - 1].at[pl.ds(0, RZ)],
                    acc.at[pl.ds(pl.multiple_of(row0 + k * RZ, 8), RZ)])

  @pl.when(s == NS - 1)
  def _zero_tail():
    pltpu.sync_copy(rows[RB - 1].at[pl.ds(0, NTAIL)],
                    acc.at[pl.ds(N - NTAIL, NTAIL)])

  plsc.subcore_barrier()

  def _scale(b):
    def edge_body(eg, ecarry):
      for u in range(UNROLL):
        e = eg * UNROLL + u
        w16 = plsc.load_gather(
            wring[b], [jnp.full((LANES,), e, jnp.int32)])
        for j in range(H // LANES):
          sl = pl.ds(j * LANES, LANES)
          rows[b][e, sl] = rows[b][e, sl] * w16
      return ecarry

    lax.fori_loop(0, C // UNROLL, edge_body, 0)

  def _visit(v, b, swait, fetch, gissue):
    nb = (b + 1) % RB
    if swait:
      _wait_scatter(nb)
    if fetch:
      _issue_fetch(v + 2, (b + 2) % RB)
    if gissue:
      _wait_fetch(nb)
      _unpack_src(nb)
      _issue_gather(nb)
    _wait_gather(b)
    _scale(b)
    _unpack_dst(b)
    _issue_scatter(b)

  _wait_fetch(0)
  _unpack_src(0)
  _issue_gather(0)
  for v in range(RB):
    _visit(v, v, swait=(v >= RB - 1), fetch=True, gissue=True)

  def ring_body(j, carry):
    v = j * RB
    for u in range(RB):
      _visit(v + u, u, swait=True, fetch=True, gissue=True)
    return carry

  lax.fori_loop(1, (NCHUNK - RB - 1) // RB, ring_body, 0)
  tail0 = ((NCHUNK - RB - 1) // RB) * RB
  for v in range(tail0, NCHUNK):
    _visit(v, v % RB, swait=True, fetch=(v + 2 < NCHUNK),
           gissue=(v + 1 < NCHUNK))
  for v in range(NCHUNK - RB + 1, NCHUNK):
    _wait_scatter(v % RB)

  plsc.subcore_barrier()
  pltpu.sync_copy(acc.at[pl.ds(row0, RPT)], out_hbm.at[c, pl.ds(row0, RPT)])

  @pl.when(s == NS - 1)
  def _write_tail():
    pltpu.sync_copy(acc.at[pl.ds(N - NTAIL, NTAIL)],
                    out_hbm.at[c, pl.ds(N - NTAIL, NTAIL)])


def _segment_sum_sc(table, src, dst, w):
  mesh = plsc.VectorSubcoreMesh(core_axis_name="c", subcore_axis_name="s",
                                num_cores=NC, num_subcores=NS)
  fn = pl.kernel(
      _seg_kernel,
      out_type=jax.ShapeDtypeStruct((NC, N, H), jnp.float32),
      mesh=mesh,
      compiler_params=pltpu.CompilerParams(needs_layout_passes=False),
      scratch_types=(
          [pltpu.VMEM((C, H), jnp.float32)] * RB
          + [pltpu.VMEM((C,), jnp.int32)] * RB
          + [pltpu.VMEM((C,), jnp.int32)] * RB
          + [pltpu.VMEM((C,), jnp.int32)] * RB
          + [pltpu.VMEM((C,), jnp.float32)] * RB
          + [pltpu.VMEM_SHARED((N, H), jnp.float32)]
          + [pltpu.SemaphoreType.DMA] * (3 * RB)
      ),
  )
  comb = jnp.left_shift(src, 14) | dst
  return fn(table, comb, w)


BM = 512
GRID = (N + BM - 1) // BM


def _mm_a_body(f_ref, f2_ref, wx0_ref, bx0_ref, wx1_ref, bx1_ref, wg0_ref,
               xh_ref, h1_ref):
  f = f_ref[...]
  h = jnp.maximum(
      jnp.dot(f, wx0_ref[...], preferred_element_type=jnp.float32)
      + bx0_ref[...], 0.0)
  xh_ref[...] = (jnp.dot(h, wx1_ref[...], preferred_element_type=jnp.float32)
                 + bx1_ref[...])
  h1_ref[...] = jnp.dot(f2_ref[...], wg0_ref[...],
                        preferred_element_type=jnp.float32)


def _mm_b_body(ap_ref, gp_ref, wa1_ref, ba1_ref, bg0_ref, wg1_ref,
               ah_ref, h2_ref):
  a = jnp.maximum(ap_ref[0] + ap_ref[1], 0.0)
  ah_ref[...] = (jnp.dot(a, wa1_ref[...], preferred_element_type=jnp.float32)
                 + ba1_ref[...])
  h1r = jnp.maximum(gp_ref[0] + gp_ref[1] + bg0_ref[...], 0.0)
  h2_ref[...] = jnp.dot(h1r, wg1_ref[...], preferred_element_type=jnp.float32)


def _mm_c_body(xh_ref, ah_ref, op_ref, bg1_ref, wz_ref, out1_ref, logp_ref):
  out1 = op_ref[0] + op_ref[1] + bg1_ref[...]
  out1_ref[...] = out1
  z = xh_ref[...] + ah_ref[...] + out1
  zz = jnp.dot(z, wz_ref[...], preferred_element_type=jnp.float32)
  m = jnp.max(zz, axis=1, keepdims=True)
  ez = jnp.exp(zz - m)
  lse = jnp.log(jnp.sum(ez, axis=1, keepdims=True)) + m
  logp_ref[...] = zz - lse


def _row_spec(shape):
  nd = len(shape)
  return pl.BlockSpec((BM,) + shape[1:], lambda i: (i,) + (0,) * (nd - 1))


def _full_spec(shape):
  nd = len(shape)
  return pl.BlockSpec(shape, lambda i: (0,) * nd)


def _part_spec(shape):
  return pl.BlockSpec((2, BM, shape[2]), lambda i: (0, i, 0))


def kernel(feature, edge_index, edge_weight, feature2, edge_index2,
           edge_weight2, Wx0, bx0, Wx1, bx1, Wa0, Wa1, ba1, Wg0, bg0, Wg1,
           bg1, Wz):
  src = edge_index[0]
  dst = edge_index[1]
  s2 = edge_index2[0]
  d2 = edge_index2[1]
  bx0r = bx0.reshape(1, H)
  bx1r = bx1.reshape(1, H)
  ba1r = ba1.reshape(1, H)
  bg0r = bg0.reshape(1, H)
  bg1r = bg1.reshape(1, H)

  a_p = _segment_sum_sc(Wa0, src, dst, edge_weight)

  x_h, h1 = pl.pallas_call(
      _mm_a_body,
      grid=(GRID,),
      in_specs=[
          _row_spec((N, IN)), _row_spec((N, IN)),
          _full_spec((IN, H)), _full_spec((1, H)),
          _full_spec((H, H)), _full_spec((1, H)),
          _full_spec((IN, H)),
      ],
      out_specs=[_row_spec((N, H)), _row_spec((N, H))],
      out_shape=[
          jax.ShapeDtypeStruct((N, H), jnp.float32),
          jax.ShapeDtypeStruct((N, H), jnp.float32),
      ],
  )(feature, feature2, Wx0, bx0r, Wx1, bx1r, Wg0)

  g_p = _segment_sum_sc(h1, s2, d2, edge_weight2)

  a_h, h2 = pl.pallas_call(
      _mm_b_body,
      grid=(GRID,),
      in_specs=[
          _part_spec((2, N, H)), _part_spec((2, N, H)),
          _full_spec((H, H)), _full_spec((1, H)),
          _full_spec((1, H)), _full_spec((H, H)),
      ],
      out_specs=[_row_spec((N, H)), _row_spec((N, H))],
      out_shape=[
          jax.ShapeDtypeStruct((N, H), jnp.float32),
          jax.ShapeDtypeStruct((N, H), jnp.float32),
      ],
  )(a_p, g_p, Wa1, ba1r, bg0r, Wg1)

  o_p = _segment_sum_sc(h2, s2, d2, edge_weight2)

  output1, logp = pl.pallas_call(
      _mm_c_body,
      grid=(GRID,),
      in_specs=[
          _row_spec((N, H)), _row_spec((N, H)), _part_spec((2, N, H)),
          _full_spec((1, H)), _full_spec((H, OUT)),
      ],
      out_specs=[_row_spec((N, H)), _row_spec((N, OUT))],
      out_shape=[
          jax.ShapeDtypeStruct((N, H), jnp.float32),
          jax.ShapeDtypeStruct((N, OUT), jnp.float32),
      ],
  )(x_h, a_h, o_p, bg1r, Wz)

  return (x_h, a_h, output1, logp)


# speedup vs baseline: 9.8838x; 1.1838x over previous
import jax
import jax.numpy as jnp
from jax import lax
from jax.experimental import pallas as pl
from jax.experimental.pallas import tpu as pltpu
from jax.experimental.pallas import tpu_sc as plsc

N = 10000
E = 320000
IN = 128
H = 128
OUT = 40

NC = 2
NS = 16
NW = NC * NS
EPW = E // NW
C = 80
NCHUNK = EPW // C
RB = 4
UNROLL = 8
RPT = 624
NTAIL = N - RPT * NS
RZ = 48
LANES = 16

_BCAST_DNUMS = jax.lax.GatherDimensionNumbers(
    offset_dims=(), collapsed_slice_dims=(0,), start_index_map=(0,))


def _sum_phase(table_hbm, comb_hbm, w_hbm, out_hbm, refs):
  rows = refs[0:RB]
  sring = refs[RB:2 * RB]
  dring = refs[2 * RB:3 * RB]
  cring = refs[3 * RB:4 * RB]
  wring = refs[4 * RB:5 * RB]
  acc = refs[5 * RB]
  gsem = refs[5 * RB + 1:6 * RB + 1]
  ssem = refs[6 * RB + 1:7 * RB + 1]
  fsem = refs[7 * RB + 1:8 * RB + 1]

  c = lax.axis_index("c")
  s = lax.axis_index("s")
  wid = c * NS + s
  ebase = pl.multiple_of(wid * EPW, 8)

  def _issue_fetch(k, r):
    off = pl.multiple_of(ebase + k * C, 8)
    pltpu.async_copy(comb_hbm.at[pl.ds(off, C)], cring[r], fsem[r])
    pltpu.async_copy(w_hbm.at[pl.ds(off, C)], wring[r], fsem[r])

  def _wait_fetch(r):
    pltpu.make_async_copy(comb_hbm.at[pl.ds(ebase, C)], cring[r],
                          fsem[r]).wait()
    pltpu.make_async_copy(w_hbm.at[pl.ds(ebase, C)], wring[r],
                          fsem[r]).wait()

  def _unpack_src(r):
    for off in range(0, C, LANES):
      sl = pl.ds(off, LANES)
      sring[r][sl] = lax.shift_right_logical(cring[r][sl], 14)

  def _unpack_dst(r):
    m = jnp.full((LANES,), 16383, jnp.int32)
    for off in range(0, C, LANES):
      sl = pl.ds(off, LANES)
      dring[r][sl] = lax.bitwise_and(cring[r][sl], m)

  def _issue_gather(b):
    return pltpu.async_copy(table_hbm.at[sring[b]], rows[b], gsem[b])

  def _issue_scatter(b):
    return pltpu.async_copy(rows[b], acc.at[dring[b]], ssem[b], add=True)

  def _wait_gather(b):
    pltpu.make_async_copy(table_hbm.at[sring[b]], rows[b], gsem[b]).wait()

  def _wait_scatter(b):
    pltpu.make_async_copy(rows[b], acc.at[dring[b]], ssem[b]).wait()

  _issue_fetch(0, 0)
  _issue_fetch(1, 1)

  zero16 = jnp.zeros((LANES,), jnp.float32)

  def zero_body(i, carry):
    for j in range(H // LANES):
      rows[RB - 1][i, pl.ds(j * LANES, LANES)] = zero16
    return carry

  lax.fori_loop(0, RZ, zero_body, 0)
  row0 = pl.multiple_of(s * RPT, 8)
  for k in range(RPT // RZ):
    pltpu.sync_copy(rows[RB - 1].at[pl.ds(0, RZ)],
                    acc.at[pl.ds(pl.multiple_of(row0 + k * RZ, 8), RZ)])

  @pl.when(s == NS - 1)
  def _zero_tail():
    pltpu.sync_copy(rows[RB - 1].at[pl.ds(0, NTAIL)],
                    acc.at[pl.ds(N - NTAIL, NTAIL)])

  plsc.subcore_barrier()

  def _scale(b):
    def group_body(g, gcarry):
      wvec = wring[b][pl.ds(g * LANES, LANES)]
      for u in range(LANES):
        e = g * LANES + u
        w16 = lax.gather(
            wvec, jnp.full((LANES, 1), u, jnp.int32), _BCAST_DNUMS, (1,),
            mode=lax.GatherScatterMode.PROMISE_IN_BOUNDS)
        for j in range(H // LANES):
          sl = pl.ds(j * LANES, LANES)
          rows[b][e, sl] = rows[b][e, sl] * w16
      return gcarry

    lax.fori_loop(0, C // LANES, group_body, 0)

  def _visit(v, b, swait, fetch, gissue):
    nb2 = (b + 2) % RB
    if swait:
      _wait_scatter(nb2)
    if fetch:
      _issue_fetch(v + 3, (b + 3) % RB)
    if gissue:
      _wait_fetch(nb2)
      _unpack_src(nb2)
      _issue_gather(nb2)
    _wait_gather(b)
    _scale(b)
    _unpack_dst(b)
    _issue_scatter(b)

  _issue_fetch(2, 2)
  _wait_fetch(0)
  _unpack_src(0)
  _issue_gather(0)
  _wait_fetch(1)
  _unpack_src(1)
  _issue_gather(1)
  for v in range(RB):
    _visit(v, v, swait=(v >= 2), fetch=True, gissue=True)

  def ring_body(j, carry):
    v = j * RB
    for u in range(RB):
      _visit(v + u, u, swait=True, fetch=True, gissue=True)
    return carry

  lax.fori_loop(1, (NCHUNK - RB - 1) // RB, ring_body, 0)
  tail0 = ((NCHUNK - RB - 1) // RB) * RB
  for v in range(tail0, NCHUNK):
    _visit(v, v % RB, swait=True, fetch=(v + 3 < NCHUNK),
           gissue=(v + 2 < NCHUNK))
  for v in range(NCHUNK - 2, NCHUNK):
    _wait_scatter(v % RB)

  plsc.subcore_barrier()
  pltpu.sync_copy(acc.at[pl.ds(row0, RPT)], out_hbm.at[c, pl.ds(row0, RPT)])

  @pl.when(s == NS - 1)
  def _write_tail():
    pltpu.sync_copy(acc.at[pl.ds(N - NTAIL, NTAIL)],
                    out_hbm.at[c, pl.ds(N - NTAIL, NTAIL)])


def _seg_kernel1(table_hbm, comb_hbm, w_hbm, out_hbm, *refs):
  _sum_phase(table_hbm, comb_hbm, w_hbm, out_hbm, refs)


_SC_SCRATCH = (
    [pltpu.VMEM((C, H), jnp.float32)] * RB
    + [pltpu.VMEM((C,), jnp.int32)] * RB
    + [pltpu.VMEM((C,), jnp.int32)] * RB
    + [pltpu.VMEM((C,), jnp.int32)] * RB
    + [pltpu.VMEM((C,), jnp.float32)] * RB
    + [pltpu.VMEM_SHARED((N, H), jnp.float32)]
    + [pltpu.SemaphoreType.DMA] * (3 * RB)
)


def _mesh():
  return plsc.VectorSubcoreMesh(core_axis_name="c", subcore_axis_name="s",
                                num_cores=NC, num_subcores=NS)


def _segment_sum_sc(table, src, dst, w):
  fn = pl.kernel(
      _seg_kernel1,
      out_type=jax.ShapeDtypeStruct((NC, N, H), jnp.float32),
      mesh=_mesh(),
      compiler_params=pltpu.CompilerParams(needs_layout_passes=False),
      scratch_types=_SC_SCRATCH,
  )
  comb = jnp.left_shift(src, 14) | dst
  return fn(table, comb, w)


def _segment_sum_sc2(t1, src1, dst1, w1, t2, src2, dst2, w2):
  fn = pl.kernel(
      _seg_kernel2,
      out_type=(jax.ShapeDtypeStruct((NC, N, H), jnp.float32),
                jax.ShapeDtypeStruct((NC, N, H), jnp.float32)),
      mesh=_mesh(),
      compiler_params=pltpu.CompilerParams(needs_layout_passes=False),
      scratch_types=_SC_SCRATCH,
  )
  comb1 = jnp.left_shift(src1, 14) | dst1
  comb2 = jnp.left_shift(src2, 14) | dst2
  return fn(t1, comb1, w1, t2, comb2, w2)


BM = 512
GRID = (N + BM - 1) // BM


def _mm_a_body(f_ref, f2_ref, wx0_ref, bx0_ref, wx1_ref, bx1_ref, wg0_ref,
               xh_ref, h1_ref):
  f = f_ref[...]
  h = jnp.maximum(
      jnp.dot(f, wx0_ref[...], preferred_element_type=jnp.float32)
      + bx0_ref[...], 0.0)
  xh_ref[...] = (jnp.dot(h, wx1_ref[...], preferred_element_type=jnp.float32)
                 + bx1_ref[...])
  h1_ref[...] = jnp.dot(f2_ref[...], wg0_ref[...],
                        preferred_element_type=jnp.float32)


def _mm_b_body(ap_ref, gp_ref, wa1_ref, ba1_ref, bg0_ref, wg1_ref,
               ah_ref, h2_ref):
  a = jnp.maximum(ap_ref[0] + ap_ref[1], 0.0)
  ah_ref[...] = (jnp.dot(a, wa1_ref[...], preferred_element_type=jnp.float32)
                 + ba1_ref[...])
  h1r = jnp.maximum(gp_ref[0] + gp_ref[1] + bg0_ref[...], 0.0)
  h2_ref[...] = jnp.dot(h1r, wg1_ref[...], preferred_element_type=jnp.float32)


def _mm_c_body(xh_ref, ah_ref, op_ref, bg1_ref, wz_ref, out1_ref, logp_ref):
  out1 = op_ref[0] + op_ref[1] + bg1_ref[...]
  out1_ref[...] = out1
  z = xh_ref[...] + ah_ref[...] + out1
  zz = jnp.dot(z, wz_ref[...], preferred_element_type=jnp.float32)
  m = jnp.max(zz, axis=1, keepdims=True)
  ez = jnp.exp(zz - m)
  lse = jnp.log(jnp.sum(ez, axis=1, keepdims=True)) + m
  logp_ref[...] = zz - lse


def _row_spec(shape):
  nd = len(shape)
  return pl.BlockSpec((BM,) + shape[1:], lambda i: (i,) + (0,) * (nd - 1))


def _full_spec(shape):
  nd = len(shape)
  return pl.BlockSpec(shape, lambda i: (0,) * nd)


def _part_spec(shape):
  return pl.BlockSpec((2, BM, shape[2]), lambda i: (0, i, 0))


def kernel(feature, edge_index, edge_weight, feature2, edge_index2,
           edge_weight2, Wx0, bx0, Wx1, bx1, Wa0, Wa1, ba1, Wg0, bg0, Wg1,
           bg1, Wz):
  src = edge_index[0]
  dst = edge_index[1]
  s2 = edge_index2[0]
  d2 = edge_index2[1]
  bx0r = bx0.reshape(1, H)
  bx1r = bx1.reshape(1, H)
  ba1r = ba1.reshape(1, H)
  bg0r = bg0.reshape(1, H)
  bg1r = bg1.reshape(1, H)

  a_p = _segment_sum_sc(Wa0, src, dst, edge_weight)

  x_h, h1 = pl.pallas_call(
      _mm_a_body,
      grid=(GRID,),
      in_specs=[
          _row_spec((N, IN)), _row_spec((N, IN)),
          _full_spec((IN, H)), _full_spec((1, H)),
          _full_spec((H, H)), _full_spec((1, H)),
          _full_spec((IN, H)),
      ],
      out_specs=[_row_spec((N, H)), _row_spec((N, H))],
      out_shape=[
          jax.ShapeDtypeStruct((N, H), jnp.float32),
          jax.ShapeDtypeStruct((N, H), jnp.float32),
      ],
  )(feature, feature2, Wx0, bx0r, Wx1, bx1r, Wg0)

  g_p = _segment_sum_sc(h1, s2, d2, edge_weight2)

  a_h, h2 = pl.pallas_call(
      _mm_b_body,
      grid=(GRID,),
      in_specs=[
          _part_spec((2, N, H)), _part_spec((2, N, H)),
          _full_spec((H, H)), _full_spec((1, H)),
          _full_spec((1, H)), _full_spec((H, H)),
      ],
      out_specs=[_row_spec((N, H)), _row_spec((N, H))],
      out_shape=[
          jax.ShapeDtypeStruct((N, H), jnp.float32),
          jax.ShapeDtypeStruct((N, H), jnp.float32),
      ],
  )(a_p, g_p, Wa1, ba1r, bg0r, Wg1)

  o_p = _segment_sum_sc(h2, s2, d2, edge_weight2)

  output1, logp = pl.pallas_call(
      _mm_c_body,
      grid=(GRID,),
      in_specs=[
          _row_spec((N, H)), _row_spec((N, H)), _part_spec((2, N, H)),
          _full_spec((1, H)), _full_spec((H, OUT)),
      ],
      out_specs=[_row_spec((N, H)), _row_spec((N, OUT))],
      out_shape=[
          jax.ShapeDtypeStruct((N, H), jnp.float32),
          jax.ShapeDtypeStruct((N, OUT), jnp.float32),
      ],
  )(x_h, a_h, o_p, bg1r, Wz)

  return (x_h, a_h, output1, logp)
